# Initial kernel scaffold; baseline (speedup 1.0000x reference)
import jax
import jax.numpy as jnp
from jax.experimental import pallas as pl


def kernel(features, batch, W, b):
    raise NotImplementedError("write your pallas kernel here")


# speedup vs baseline: 251.2010x; 251.2010x over previous
import functools

import jax
import jax.numpy as jnp
from jax.experimental import pallas as pl
import jax.experimental.pallas.tpu as pltpu

N = 4096
D = 512
K = 16
NBAGS = 16
RT = 512
CT = 512
NRT = N // RT
NCT = N // CT

_DOT = dict(preferred_element_type=jnp.float32)


def _k1_norm_xw(f_ref, w_ref, nf_ref, xw_ref):
    f = f_ref[...]
    nrm = jnp.sqrt(jnp.sum(f * f, axis=1, keepdims=True))
    nf_ref[...] = (f / jnp.maximum(nrm, 1e-12)).astype(jnp.bfloat16)
    xw_ref[...] = jax.lax.dot_general(f.astype(jnp.bfloat16),
                                      w_ref[...].astype(jnp.bfloat16),
                                      (((1,), (0,)), ((), ())), **_DOT)


def _k2_corr_topk(nf_t_ref, nf_all_ref, bcol_ref, brow_ref,
                  adj_ref, deg_ref):
    i = pl.program_id(0)
    v = jax.lax.dot_general(nf_t_ref[...], nf_all_ref[...],
                            (((1,), (1,)), ((), ())), **_DOT)
    mask = bcol_ref[...] == brow_ref[...]
    v = jnp.where(mask, v, 0.0)
    t = jnp.max(v, axis=1, keepdims=True)
    c = jnp.sum((v >= t).astype(jnp.float32), axis=1, keepdims=True)
    for _ in range(K - 1):
        nt = jnp.max(jnp.where(v < t, v, -jnp.inf), axis=1, keepdims=True)
        t = jnp.where(c < K, nt, t)
        c = jnp.sum((v >= t).astype(jnp.float32), axis=1, keepdims=True)
    adj = jnp.where(v >= t, v, 0.0)
    adj_ref[...] = adj
    row_id = i * RT + jax.lax.broadcasted_iota(jnp.int32, (RT, N), 0)
    col_id = jax.lax.broadcasted_iota(jnp.int32, (RT, N), 1)
    a_sl = ((adj != 0.0) | (row_id == col_id)).astype(jnp.float32)
    deg_ref[...] = jnp.sum(a_sl, axis=0, keepdims=True)[None]


def _k3_agg(adj_ref, xw_ref, dinv_ref, bcol_ref, b_ref, agg_ref, acc_ref):
    ct = pl.program_id(0)
    rt = pl.program_id(1)

    @pl.when(jnp.logical_and(ct == 0, rt == 0))
    def _():
        agg_ref[...] = jnp.zeros_like(agg_ref)

    @pl.when(rt == 0)
    def _():
        acc_ref[...] = jnp.zeros_like(acc_ref)

    a = adj_ref[...]
    row_id = rt * CT + jax.lax.broadcasted_iota(jnp.int32, (CT, CT), 0)
    col_id = ct * CT + jax.lax.broadcasted_iota(jnp.int32, (CT, CT), 1)
    a_sl = ((a != 0.0) | (row_id == col_id)).astype(jnp.bfloat16)
    msg = (xw_ref[pl.ds(rt * CT, CT), :] *
           dinv_ref[pl.ds(rt * CT, CT), :]).astype(jnp.bfloat16)
    acc_ref[...] += jax.lax.dot_general(a_sl, msg, (((0,), (0,)), ((), ())),
                                        **_DOT)

    @pl.when(rt == NRT - 1)
    def _():
        out = dinv_ref[pl.ds(ct * CT, CT), :] * acc_ref[...] + b_ref[...]
        bt = bcol_ref[pl.ds(ct * CT, CT), :]
        bags = jax.lax.broadcasted_iota(jnp.int32, (CT, NBAGS), 1).astype(
            jnp.float32)
        onehot = (bt == bags).astype(jnp.bfloat16)
        agg_ref[...] += jax.lax.dot_general(onehot, out.astype(jnp.bfloat16),
                                            (((0,), (0,)), ((), ())), **_DOT)


@jax.jit
def kernel(features, batch, W, b):
    bcol = batch.astype(jnp.float32).reshape(N, 1)
    brow = batch.astype(jnp.float32).reshape(1, N)
    b_row = b.reshape(1, D)

    nf, xw = pl.pallas_call(
        _k1_norm_xw,
        grid=(NRT,),
        in_specs=[
            pl.BlockSpec((RT, D), lambda i: (i, 0)),
            pl.BlockSpec((D, D), lambda i: (0, 0)),
        ],
        out_specs=[
            pl.BlockSpec((RT, D), lambda i: (i, 0)),
            pl.BlockSpec((RT, D), lambda i: (i, 0)),
        ],
        out_shape=[
            jax.ShapeDtypeStruct((N, D), jnp.bfloat16),
            jax.ShapeDtypeStruct((N, D), jnp.float32),
        ],
    )(features, W)

    adj, degp = pl.pallas_call(
        _k2_corr_topk,
        grid=(NRT,),
        in_specs=[
            pl.BlockSpec((RT, D), lambda i: (i, 0)),
            pl.BlockSpec((N, D), lambda i: (0, 0)),
            pl.BlockSpec((RT, 1), lambda i: (i, 0)),
            pl.BlockSpec((1, N), lambda i: (0, 0)),
        ],
        out_specs=[
            pl.BlockSpec((RT, N), lambda i: (i, 0)),
            pl.BlockSpec((1, 1, N), lambda i: (i, 0, 0)),
        ],
        out_shape=[
            jax.ShapeDtypeStruct((N, N), jnp.float32),
            jax.ShapeDtypeStruct((NRT, 1, N), jnp.float32),
        ],
    )(nf, nf, bcol, brow)

    deg = jnp.sum(degp.reshape(NRT, N), axis=0)
    dinv = jnp.where(deg > 0.0, jax.lax.rsqrt(deg), 0.0).reshape(N, 1)

    agg = pl.pallas_call(
        _k3_agg,
        grid=(NCT, NRT),
        in_specs=[
            pl.BlockSpec((CT, CT), lambda ct, rt: (rt, ct)),
            pl.BlockSpec((N, D), lambda ct, rt: (0, 0)),
            pl.BlockSpec((N, 1), lambda ct, rt: (0, 0)),
            pl.BlockSpec((N, 1), lambda ct, rt: (0, 0)),
            pl.BlockSpec((1, D), lambda ct, rt: (0, 0)),
        ],
        out_specs=pl.BlockSpec((NBAGS, D), lambda ct, rt: (0, 0)),
        out_shape=jax.ShapeDtypeStruct((NBAGS, D), jnp.float32),
        scratch_shapes=[pltpu.VMEM((CT, D), jnp.float32)],
    )(adj, xw, dinv, bcol, b_row)

    return agg, adj
